# baseline (speedup 1.0000x reference)
import functools

import jax
import jax.numpy as jnp
from jax import lax
from jax.experimental import pallas as pl
from jax.experimental.pallas import tpu as pltpu
from jax.experimental.pallas import tpu_sc as plsc

_ROWS = 16384
_COLS = 128
_NC = 2
_NS = 16
_NW = _NC * _NS
_RPW = _ROWS // _NW


def _sc_body(x_hbm, o_hbm, rows_v):
    wid = lax.axis_index("s") * _NC + lax.axis_index("c")
    base = wid * _RPW
    pltpu.sync_copy(x_hbm.at[pl.ds(base, _RPW), :], rows_v)

    @pl.when(wid == 0)
    def _():
        lane = lax.iota(jnp.int32, 16)
        head = rows_v[0, 0:16]
        rows_v[0, 0:16] = jnp.where(lane == 0, 3.0, head)

    pltpu.sync_copy(rows_v, o_hbm.at[pl.ds(base, _RPW), :])


def kernel(x):
    mesh = plsc.VectorSubcoreMesh(core_axis_name="c", subcore_axis_name="s")
    k = functools.partial(
        pl.kernel,
        out_type=jax.ShapeDtypeStruct((_ROWS, _COLS), jnp.float32),
        mesh=mesh,
        scratch_types=[pltpu.VMEM((_RPW, _COLS), jnp.float32)],
    )(_sc_body)
    return k(x)


# speedup vs baseline: 3.6825x; 3.6825x over previous
import jax
import jax.numpy as jnp
from jax.experimental import pallas as pl
from jax.experimental.pallas import tpu as pltpu

_ROWS = 16384
_COLS = 128
_CHUNK = 2048
_NCHUNKS = _ROWS // _CHUNK
_WINDOW = 4


def _copy_set_kernel(x_hbm, o_hbm, scratch, in_sems, out_sems):
    def in_copy(c):
        return pltpu.make_async_copy(
            x_hbm.at[pl.ds(c * _CHUNK, _CHUNK), :],
            scratch.at[c], in_sems.at[c])

    def out_copy(c):
        return pltpu.make_async_copy(
            scratch.at[c],
            o_hbm.at[pl.ds(c * _CHUNK, _CHUNK), :], out_sems.at[c])

    for c in range(_WINDOW):
        in_copy(c).start()
    for c in range(_NCHUNKS):
        in_copy(c).wait()
        if c == 0:
            col = jax.lax.broadcasted_iota(jnp.int32, (1, _COLS), 1)
            scratch[0, 0:1, :] = jnp.where(col == 0, 3.0, scratch[0, 0:1, :])
        out_copy(c).start()
        if c + _WINDOW < _NCHUNKS:
            in_copy(c + _WINDOW).start()
    for c in range(_NCHUNKS):
        out_copy(c).wait()


def kernel(x):
    return pl.pallas_call(
        _copy_set_kernel,
        in_specs=[pl.BlockSpec(memory_space=pl.ANY)],
        out_specs=pl.BlockSpec(memory_space=pl.ANY),
        out_shape=jax.ShapeDtypeStruct((_ROWS, _COLS), jnp.float32),
        scratch_shapes=[
            pltpu.VMEM((_NCHUNKS, _CHUNK, _COLS), jnp.float32),
            pltpu.SemaphoreType.DMA((_NCHUNKS,)),
            pltpu.SemaphoreType.DMA((_NCHUNKS,)),
        ],
    )(x)


# speedup vs baseline: 4.2313x; 1.1490x over previous
import jax
import jax.numpy as jnp
from jax.experimental import pallas as pl
from jax.experimental.pallas import tpu as pltpu

_ROWS = 16384
_COLS = 128
_CHUNK = 2048
_NSLOTS = 8
_NCHUNKS = _ROWS // _CHUNK


def _copy_set_kernel(x_hbm, o_hbm, scratch, in_sems, out_sems):
    def in_copy(c):
        slot = c % _NSLOTS
        return pltpu.make_async_copy(
            x_hbm.at[pl.ds(c * _CHUNK, _CHUNK), :],
            scratch.at[slot], in_sems.at[slot])

    def out_copy(c):
        slot = c % _NSLOTS
        return pltpu.make_async_copy(
            scratch.at[slot],
            o_hbm.at[pl.ds(c * _CHUNK, _CHUNK), :], out_sems.at[slot])

    for c in range(min(_NSLOTS, _NCHUNKS)):
        in_copy(c).start()
    for c in range(_NCHUNKS):
        in_copy(c).wait()
        if c == 0:
            col = jax.lax.broadcasted_iota(jnp.int32, (1, _COLS), 1)
            scratch[0, 0:1, :] = jnp.where(col == 0, 3.0, scratch[0, 0:1, :])
        out_copy(c).start()
        nxt = c + _NSLOTS
        if nxt < _NCHUNKS:
            out_copy(c).wait()
            in_copy(nxt).start()
    for c in range(max(_NCHUNKS - _NSLOTS, 0), _NCHUNKS):
        out_copy(c).wait()


def kernel(x):
    return pl.pallas_call(
        _copy_set_kernel,
        in_specs=[pl.BlockSpec(memory_space=pl.ANY)],
        out_specs=pl.BlockSpec(memory_space=pl.ANY),
        out_shape=jax.ShapeDtypeStruct((_ROWS, _COLS), jnp.float32),
        scratch_shapes=[
            pltpu.VMEM((_NSLOTS, _CHUNK, _COLS), jnp.float32),
            pltpu.SemaphoreType.DMA((_NSLOTS,)),
            pltpu.SemaphoreType.DMA((_NSLOTS,)),
        ],
    )(x)
